# speedup vs baseline: 1.0778x; 1.0778x over previous
import functools

import jax
import jax.numpy as jnp
from jax.experimental import pallas as pl

N = 10000
E = 320000
NODE_F = 128
EDGE_F = 16
H = 256
HEADS = 4
NUM_CLASSES = 10
NUM_CLUSTERS = 10

BN = 1000
BE = 2000


def _ln_rows(y, g, b):
    m = jnp.mean(y, axis=-1, keepdims=True)
    v = jnp.mean((y - m) ** 2, axis=-1, keepdims=True)
    return (y - m) * jax.lax.rsqrt(v + 1e-5) * g + b


def _enc_body(x_ref, w1_ref, b1_ref, w2_ref, b2_ref, g_ref, be_ref, o_ref):
    h1 = jnp.maximum(
        jnp.dot(x_ref[...], w1_ref[...], preferred_element_type=jnp.float32)
        + b1_ref[...], 0.0)
    h2 = jnp.dot(h1, w2_ref[...], preferred_element_type=jnp.float32) + b2_ref[...]
    o_ref[...] = _ln_rows(h2, g_ref[...], be_ref[...])


def _encoder(x, p):
    return pl.pallas_call(
        _enc_body,
        grid=(N // BN,),
        in_specs=[
            pl.BlockSpec((BN, NODE_F), lambda i: (i, 0)),
            pl.BlockSpec((NODE_F, H), lambda i: (0, 0)),
            pl.BlockSpec((H,), lambda i: (0,)),
            pl.BlockSpec((H, H), lambda i: (0, 0)),
            pl.BlockSpec((H,), lambda i: (0,)),
            pl.BlockSpec((H,), lambda i: (0,)),
            pl.BlockSpec((H,), lambda i: (0,)),
        ],
        out_specs=pl.BlockSpec((BN, H), lambda i: (i, 0)),
        out_shape=jax.ShapeDtypeStruct((N, H), jnp.float32),
    )(x, p['ne_w1'], p['ne_b1'], p['ne_w2'], p['ne_b2'], p['ne_g'], p['ne_be'])


def _gat_proj_body(h_ref, w_ref, as_ref, ad_ref, hw_ref, sd_ref):
    hw = jnp.dot(h_ref[...], w_ref[...], preferred_element_type=jnp.float32)
    hw_ref[...] = hw
    cols = []
    for hd in range(HEADS):
        blk = hw[:, hd * H:(hd + 1) * H]
        cols.append(jnp.sum(blk * as_ref[hd, :], axis=-1, keepdims=True))
    for hd in range(HEADS):
        blk = hw[:, hd * H:(hd + 1) * H]
        cols.append(jnp.sum(blk * ad_ref[hd, :], axis=-1, keepdims=True))
    sd_ref[...] = jnp.concatenate(cols, axis=-1)


def _gat_project(h, w, a_s, a_d):
    k = h.shape[1]
    return pl.pallas_call(
        _gat_proj_body,
        grid=(N // BN,),
        in_specs=[
            pl.BlockSpec((BN, k), lambda i: (i, 0)),
            pl.BlockSpec((k, HEADS * H), lambda i: (0, 0)),
            pl.BlockSpec((HEADS, H), lambda i: (0, 0)),
            pl.BlockSpec((HEADS, H), lambda i: (0, 0)),
        ],
        out_specs=[
            pl.BlockSpec((BN, HEADS * H), lambda i: (i, 0)),
            pl.BlockSpec((BN, 2 * HEADS), lambda i: (i, 0)),
        ],
        out_shape=[
            jax.ShapeDtypeStruct((N, HEADS * H), jnp.float32),
            jax.ShapeDtypeStruct((N, 2 * HEADS), jnp.float32),
        ],
    )(h, w, a_s, a_d)


def _gat_post_body(agg_ref, den_ref, hw_ref, sd_ref, b_ref, g_ref, be_ref,
                   o_ref, *, concat):
    sd = sd_ref[...]
    e_self = sd[:, :HEADS] + sd[:, HEADS:]
    ee_self = jnp.exp(jnp.where(e_self > 0, e_self, 0.2 * e_self))
    hw = hw_ref[...]
    agg = agg_ref[...]
    den = den_ref[...] + ee_self
    outs = []
    for hd in range(HEADS):
        a = (agg[:, hd * H:(hd + 1) * H]
             + ee_self[:, hd:hd + 1] * hw[:, hd * H:(hd + 1) * H])
        outs.append(a / (den[:, hd:hd + 1] + 1e-16))
    if concat:
        y = jnp.concatenate(outs, axis=-1) + b_ref[...]
    else:
        y = (outs[0] + outs[1] + outs[2] + outs[3]) * 0.25 + b_ref[...]
    y = _ln_rows(y, g_ref[...], be_ref[...])
    o_ref[...] = jnp.where(y > 0, y, jnp.exp(jnp.minimum(y, 0.0)) - 1.0)


def _gat_post(agg, den, hw, sd, b, g, be, concat):
    od = HEADS * H if concat else H
    return pl.pallas_call(
        functools.partial(_gat_post_body, concat=concat),
        grid=(N // BN,),
        in_specs=[
            pl.BlockSpec((BN, HEADS * H), lambda i: (i, 0)),
            pl.BlockSpec((BN, HEADS), lambda i: (i, 0)),
            pl.BlockSpec((BN, HEADS * H), lambda i: (i, 0)),
            pl.BlockSpec((BN, 2 * HEADS), lambda i: (i, 0)),
            pl.BlockSpec((od,), lambda i: (0,)),
            pl.BlockSpec((od,), lambda i: (0,)),
            pl.BlockSpec((od,), lambda i: (0,)),
        ],
        out_specs=pl.BlockSpec((BN, od), lambda i: (i, 0)),
        out_shape=jax.ShapeDtypeStruct((N, od), jnp.float32),
    )(agg, den, hw, sd, b, g, be)


def _final_dense_body(h_ref, caw1_ref, cab1_ref, caw2_ref, cab2_ref,
                      ncw1_ref, ncb1_ref, ncw2_ref, ncb2_ref,
                      w1a_ref, w1b_ref, gcs_ref,
                      cp_ref, nl_ref, a_ref, b_ref, hs_ref):
    h = h_ref[...]
    t = jnp.maximum(jnp.dot(h, caw1_ref[...], preferred_element_type=jnp.float32)
                    + cab1_ref[...], 0.0)
    logits = (jnp.dot(t, caw2_ref[...], preferred_element_type=jnp.float32)
              + cab2_ref[...])
    lm = jnp.max(logits, axis=-1, keepdims=True)
    el = jnp.exp(logits - lm)
    cp_ref[...] = el / jnp.sum(el, axis=-1, keepdims=True)
    t2 = jnp.maximum(jnp.dot(h, ncw1_ref[...], preferred_element_type=jnp.float32)
                     + ncb1_ref[...], 0.0)
    nl_ref[...] = (jnp.dot(t2, ncw2_ref[...], preferred_element_type=jnp.float32)
                   + ncb2_ref[...])
    a_ref[...] = jnp.dot(h, w1a_ref[...], preferred_element_type=jnp.float32)
    b_ref[...] = jnp.dot(h, w1b_ref[...], preferred_element_type=jnp.float32)
    hs_ref[...] = jnp.dot(h, gcs_ref[...], preferred_element_type=jnp.float32)


def _final_dense(h, p):
    w1a = p['ea_w1'][:H]
    w1b = p['ea_w1'][H:2 * H]
    return pl.pallas_call(
        _final_dense_body,
        grid=(N // BN,),
        in_specs=[
            pl.BlockSpec((BN, H), lambda i: (i, 0)),
            pl.BlockSpec((H, H), lambda i: (0, 0)),
            pl.BlockSpec((H,), lambda i: (0,)),
            pl.BlockSpec((H, NUM_CLUSTERS), lambda i: (0, 0)),
            pl.BlockSpec((NUM_CLUSTERS,), lambda i: (0,)),
            pl.BlockSpec((H, H // 2), lambda i: (0, 0)),
            pl.BlockSpec((H // 2,), lambda i: (0,)),
            pl.BlockSpec((H // 2, NUM_CLASSES), lambda i: (0, 0)),
            pl.BlockSpec((NUM_CLASSES,), lambda i: (0,)),
            pl.BlockSpec((H, H), lambda i: (0, 0)),
            pl.BlockSpec((H, H), lambda i: (0, 0)),
            pl.BlockSpec((H, H), lambda i: (0, 0)),
        ],
        out_specs=[
            pl.BlockSpec((BN, NUM_CLUSTERS), lambda i: (i, 0)),
            pl.BlockSpec((BN, NUM_CLASSES), lambda i: (i, 0)),
            pl.BlockSpec((BN, H), lambda i: (i, 0)),
            pl.BlockSpec((BN, H), lambda i: (i, 0)),
            pl.BlockSpec((BN, H), lambda i: (i, 0)),
        ],
        out_shape=[
            jax.ShapeDtypeStruct((N, NUM_CLUSTERS), jnp.float32),
            jax.ShapeDtypeStruct((N, NUM_CLASSES), jnp.float32),
            jax.ShapeDtypeStruct((N, H), jnp.float32),
            jax.ShapeDtypeStruct((N, H), jnp.float32),
            jax.ShapeDtypeStruct((N, H), jnp.float32),
        ],
    )(h, p['ca_w1'], p['ca_b1'], p['ca_w2'], p['ca_b2'],
      p['nc_w1'], p['nc_b1'], p['nc_w2'], p['nc_b2'],
      w1a, w1b, p['gcs_w'])


def _conv_pool_body(neigh_ref, hs_ref, gcr_ref, gcrb_ref,
                    conv_ref, psum_ref, pmax_ref):
    i = pl.program_id(0)
    conv = (jnp.dot(neigh_ref[...], gcr_ref[...], preferred_element_type=jnp.float32)
            + gcrb_ref[...] + hs_ref[...])
    conv_ref[...] = conv
    bsum = jnp.sum(conv, axis=0, keepdims=True)
    bmax = jnp.max(conv, axis=0, keepdims=True)

    @pl.when(i == 0)
    def _():
        psum_ref[...] = bsum
        pmax_ref[...] = bmax

    @pl.when(i > 0)
    def _():
        psum_ref[...] += bsum
        pmax_ref[...] = jnp.maximum(pmax_ref[...], bmax)


def _conv_pool(neigh, hs, p):
    return pl.pallas_call(
        _conv_pool_body,
        grid=(N // BN,),
        in_specs=[
            pl.BlockSpec((BN, H), lambda i: (i, 0)),
            pl.BlockSpec((BN, H), lambda i: (i, 0)),
            pl.BlockSpec((H, H), lambda i: (0, 0)),
            pl.BlockSpec((H,), lambda i: (0,)),
        ],
        out_specs=[
            pl.BlockSpec((BN, H), lambda i: (i, 0)),
            pl.BlockSpec((1, H), lambda i: (0, 0)),
            pl.BlockSpec((1, H), lambda i: (0, 0)),
        ],
        out_shape=[
            jax.ShapeDtypeStruct((N, H), jnp.float32),
            jax.ShapeDtypeStruct((1, H), jnp.float32),
            jax.ShapeDtypeStruct((1, H), jnp.float32),
        ],
    )(neigh, hs, p['gcr_w'], p['gcr_b'])


def _graph_head_body(psum_ref, pmax_ref, opw_ref, opb_ref,
                     gw1_ref, gb1_ref, gw2_ref, gb2_ref, o_ref):
    add_p = psum_ref[...]
    mean_p = add_p / float(N)
    max_p = pmax_ref[...]
    ge = (jnp.dot(mean_p, opw_ref[0], preferred_element_type=jnp.float32)
          + jnp.dot(max_p, opw_ref[1], preferred_element_type=jnp.float32)
          + jnp.dot(add_p, opw_ref[2], preferred_element_type=jnp.float32)
          + opb_ref[...])
    t = jnp.maximum(jnp.dot(ge, gw1_ref[...], preferred_element_type=jnp.float32)
                    + gb1_ref[...], 0.0)
    o_ref[...] = (jnp.dot(t, gw2_ref[...], preferred_element_type=jnp.float32)
                  + gb2_ref[...])


def _graph_head(psum, pmax, p):
    opw = p['op_w'].reshape(3, H, H)
    return pl.pallas_call(
        _graph_head_body,
        out_shape=jax.ShapeDtypeStruct((1, 2), jnp.float32),
    )(psum, pmax, opw, p['op_b'], p['gc_w1'], p['gc_b1'], p['gc_w2'], p['gc_b2'])


def _edge_score_body(ab_ref, ea_ref, wc_ref, b1_ref, w2_ref, b2_ref, o_ref):
    u = (ab_ref[...]
         + jnp.dot(ea_ref[...], wc_ref[...], preferred_element_type=jnp.float32)
         + b1_ref[...])
    u = jnp.maximum(u, 0.0)
    s = jnp.dot(u, w2_ref[...], preferred_element_type=jnp.float32) + b2_ref[...]
    o_ref[...] = jax.nn.sigmoid(s)


def _edge_scores(ab, edge_attr, p):
    wc = p['ea_w1'][2 * H:]
    return pl.pallas_call(
        _edge_score_body,
        grid=(E // BE,),
        in_specs=[
            pl.BlockSpec((BE, H), lambda i: (i, 0)),
            pl.BlockSpec((BE, EDGE_F), lambda i: (i, 0)),
            pl.BlockSpec((EDGE_F, H), lambda i: (0, 0)),
            pl.BlockSpec((H,), lambda i: (0,)),
            pl.BlockSpec((H, 1), lambda i: (0, 0)),
            pl.BlockSpec((1,), lambda i: (0,)),
        ],
        out_specs=pl.BlockSpec((BE, 1), lambda i: (i, 0)),
        out_shape=jax.ShapeDtypeStruct((E, 1), jnp.float32),
    )(ab, edge_attr, wc, p['ea_b1'], p['ea_w2'], p['ea_b2'])


def kernel(x, edge_index, edge_attr, batch, params):
    p = params
    src = edge_index[0]
    dst = edge_index[1]

    h = _encoder(x, p)

    gat_concat = (True, True, False)
    for i in range(3):
        hw, sd = _gat_project(h, p['gat%d_w' % i], p['gat%d_as' % i],
                              p['gat%d_ad' % i])
        s_e = jnp.take(sd[:, :HEADS], src, axis=0)
        d_e = jnp.take(sd[:, HEADS:], dst, axis=0)
        e = s_e + d_e
        ee = jnp.exp(jnp.where(e > 0, e, 0.2 * e))
        den = jax.ops.segment_sum(ee, dst, num_segments=N)
        hw3 = hw.reshape(N, HEADS, H)
        msg = jnp.take(hw3, src, axis=0) * ee[:, :, None]
        agg = jax.ops.segment_sum(msg, dst, num_segments=N).reshape(N, HEADS * H)
        h = _gat_post(agg, den, hw, sd, p['gat%d_b' % i], p['nrm%d_g' % i],
                      p['nrm%d_b' % i], gat_concat[i])

    cp, node_logits, A, B, hs = _final_dense(h, p)

    neigh = jax.ops.segment_sum(jnp.take(h, src, axis=0), dst, num_segments=N)
    conv, psum, pmax = _conv_pool(neigh, hs, p)
    graph_logits = _graph_head(psum, pmax, p)

    ab = jnp.take(A, src, axis=0) + jnp.take(B, dst, axis=0)
    edge_scores = _edge_scores(ab, edge_attr, p)

    return (node_logits, graph_logits, edge_scores, cp)


# speedup vs baseline: 10.9625x; 10.1713x over previous
import functools

import jax
import jax.numpy as jnp
from jax import lax
from jax.experimental import pallas as pl
from jax.experimental.pallas import tpu as pltpu
from jax.experimental.pallas import tpu_sc as plsc

N = 10000
E = 320000
NODE_F = 128
EDGE_F = 16
H = 256
HEADS = 4
NUM_CLASSES = 10
NUM_CLUSTERS = 10

BN = 1000
BE = 2000


def _ln_rows(y, g, b):
    m = jnp.mean(y, axis=-1, keepdims=True)
    v = jnp.mean((y - m) ** 2, axis=-1, keepdims=True)
    return (y - m) * jax.lax.rsqrt(v + 1e-5) * g + b


def _enc_body(x_ref, w1_ref, b1_ref, w2_ref, b2_ref, g_ref, be_ref, o_ref):
    h1 = jnp.maximum(
        jnp.dot(x_ref[...], w1_ref[...], preferred_element_type=jnp.float32)
        + b1_ref[...], 0.0)
    h2 = jnp.dot(h1, w2_ref[...], preferred_element_type=jnp.float32) + b2_ref[...]
    o_ref[...] = _ln_rows(h2, g_ref[...], be_ref[...])


def _encoder(x, p):
    return pl.pallas_call(
        _enc_body,
        grid=(N // BN,),
        in_specs=[
            pl.BlockSpec((BN, NODE_F), lambda i: (i, 0)),
            pl.BlockSpec((NODE_F, H), lambda i: (0, 0)),
            pl.BlockSpec((H,), lambda i: (0,)),
            pl.BlockSpec((H, H), lambda i: (0, 0)),
            pl.BlockSpec((H,), lambda i: (0,)),
            pl.BlockSpec((H,), lambda i: (0,)),
            pl.BlockSpec((H,), lambda i: (0,)),
        ],
        out_specs=pl.BlockSpec((BN, H), lambda i: (i, 0)),
        out_shape=jax.ShapeDtypeStruct((N, H), jnp.float32),
    )(x, p['ne_w1'], p['ne_b1'], p['ne_w2'], p['ne_b2'], p['ne_g'], p['ne_be'])


def _gat_proj_body(h_ref, w_ref, as_ref, ad_ref, hw_ref, sd_ref):
    hw = jnp.dot(h_ref[...], w_ref[...], preferred_element_type=jnp.float32)
    hw_ref[...] = hw
    cols = []
    for hd in range(HEADS):
        blk = hw[:, hd * H:(hd + 1) * H]
        cols.append(jnp.sum(blk * as_ref[hd, :], axis=-1, keepdims=True))
    for hd in range(HEADS):
        blk = hw[:, hd * H:(hd + 1) * H]
        cols.append(jnp.sum(blk * ad_ref[hd, :], axis=-1, keepdims=True))
    sd_ref[...] = jnp.concatenate(cols, axis=-1)


def _gat_project(h, w, a_s, a_d):
    k = h.shape[1]
    return pl.pallas_call(
        _gat_proj_body,
        grid=(N // BN,),
        in_specs=[
            pl.BlockSpec((BN, k), lambda i: (i, 0)),
            pl.BlockSpec((k, HEADS * H), lambda i: (0, 0)),
            pl.BlockSpec((HEADS, H), lambda i: (0, 0)),
            pl.BlockSpec((HEADS, H), lambda i: (0, 0)),
        ],
        out_specs=[
            pl.BlockSpec((BN, HEADS * H), lambda i: (i, 0)),
            pl.BlockSpec((BN, 2 * HEADS), lambda i: (i, 0)),
        ],
        out_shape=[
            jax.ShapeDtypeStruct((N, HEADS * H), jnp.float32),
            jax.ShapeDtypeStruct((N, 2 * HEADS), jnp.float32),
        ],
    )(h, w, a_s, a_d)


def _gat_post_body(agg_ref, den_ref, hw_ref, sd_ref, b_ref, g_ref, be_ref,
                   o_ref, *, concat):
    sd = sd_ref[...]
    e_self = sd[:, :HEADS] + sd[:, HEADS:]
    ee_self = jnp.exp(jnp.where(e_self > 0, e_self, 0.2 * e_self))
    hw = hw_ref[...]
    den = den_ref[0] + den_ref[1] + ee_self
    outs = []
    for hd in range(HEADS):
        agg_hd = jnp.concatenate([agg_ref[2 * hd], agg_ref[2 * hd + 1]],
                                 axis=-1)
        a = (agg_hd
             + ee_self[:, hd:hd + 1] * hw[:, hd * H:(hd + 1) * H])
        outs.append(a / (den[:, hd:hd + 1] + 1e-16))
    if concat:
        y = jnp.concatenate(outs, axis=-1) + b_ref[...]
    else:
        y = (outs[0] + outs[1] + outs[2] + outs[3]) * 0.25 + b_ref[...]
    y = _ln_rows(y, g_ref[...], be_ref[...])
    o_ref[...] = jnp.where(y > 0, y, jnp.exp(jnp.minimum(y, 0.0)) - 1.0)


def _gat_post(agg, den, hw, sd, b, g, be, concat):
    od = HEADS * H if concat else H
    return pl.pallas_call(
        functools.partial(_gat_post_body, concat=concat),
        grid=(N // BN,),
        in_specs=[
            pl.BlockSpec((8, BN, 128), lambda i: (0, i, 0)),
            pl.BlockSpec((_NC, BN, HEADS), lambda i: (0, i, 0)),
            pl.BlockSpec((BN, HEADS * H), lambda i: (i, 0)),
            pl.BlockSpec((BN, 2 * HEADS), lambda i: (i, 0)),
            pl.BlockSpec((od,), lambda i: (0,)),
            pl.BlockSpec((od,), lambda i: (0,)),
            pl.BlockSpec((od,), lambda i: (0,)),
        ],
        out_specs=pl.BlockSpec((BN, od), lambda i: (i, 0)),
        out_shape=jax.ShapeDtypeStruct((N, od), jnp.float32),
    )(agg, den, hw, sd, b, g, be)


def _final_dense_body(h_ref, caw1_ref, cab1_ref, caw2_ref, cab2_ref,
                      ncw1_ref, ncb1_ref, ncw2_ref, ncb2_ref,
                      w1a_ref, w1b_ref, gcs_ref,
                      cp_ref, nl_ref, a_ref, b_ref, hs_ref):
    h = h_ref[...]
    t = jnp.maximum(jnp.dot(h, caw1_ref[...], preferred_element_type=jnp.float32)
                    + cab1_ref[...], 0.0)
    logits = (jnp.dot(t, caw2_ref[...], preferred_element_type=jnp.float32)
              + cab2_ref[...])
    lm = jnp.max(logits, axis=-1, keepdims=True)
    el = jnp.exp(logits - lm)
    cp_ref[...] = el / jnp.sum(el, axis=-1, keepdims=True)
    t2 = jnp.maximum(jnp.dot(h, ncw1_ref[...], preferred_element_type=jnp.float32)
                     + ncb1_ref[...], 0.0)
    nl_ref[...] = (jnp.dot(t2, ncw2_ref[...], preferred_element_type=jnp.float32)
                   + ncb2_ref[...])
    a_ref[...] = jnp.dot(h, w1a_ref[...], preferred_element_type=jnp.float32)
    b_ref[...] = jnp.dot(h, w1b_ref[...], preferred_element_type=jnp.float32)
    hs_ref[...] = jnp.dot(h, gcs_ref[...], preferred_element_type=jnp.float32)


def _final_dense(h, p):
    w1a = p['ea_w1'][:H]
    w1b = p['ea_w1'][H:2 * H]
    return pl.pallas_call(
        _final_dense_body,
        grid=(N // BN,),
        in_specs=[
            pl.BlockSpec((BN, H), lambda i: (i, 0)),
            pl.BlockSpec((H, H), lambda i: (0, 0)),
            pl.BlockSpec((H,), lambda i: (0,)),
            pl.BlockSpec((H, NUM_CLUSTERS), lambda i: (0, 0)),
            pl.BlockSpec((NUM_CLUSTERS,), lambda i: (0,)),
            pl.BlockSpec((H, H // 2), lambda i: (0, 0)),
            pl.BlockSpec((H // 2,), lambda i: (0,)),
            pl.BlockSpec((H // 2, NUM_CLASSES), lambda i: (0, 0)),
            pl.BlockSpec((NUM_CLASSES,), lambda i: (0,)),
            pl.BlockSpec((H, H), lambda i: (0, 0)),
            pl.BlockSpec((H, H), lambda i: (0, 0)),
            pl.BlockSpec((H, H), lambda i: (0, 0)),
        ],
        out_specs=[
            pl.BlockSpec((BN, NUM_CLUSTERS), lambda i: (i, 0)),
            pl.BlockSpec((BN, NUM_CLASSES), lambda i: (i, 0)),
            pl.BlockSpec((BN, H), lambda i: (i, 0)),
            pl.BlockSpec((BN, H), lambda i: (i, 0)),
            pl.BlockSpec((BN, H), lambda i: (i, 0)),
        ],
        out_shape=[
            jax.ShapeDtypeStruct((N, NUM_CLUSTERS), jnp.float32),
            jax.ShapeDtypeStruct((N, NUM_CLASSES), jnp.float32),
            jax.ShapeDtypeStruct((N, H), jnp.float32),
            jax.ShapeDtypeStruct((N, H), jnp.float32),
            jax.ShapeDtypeStruct((N, H), jnp.float32),
        ],
    )(h, p['ca_w1'], p['ca_b1'], p['ca_w2'], p['ca_b2'],
      p['nc_w1'], p['nc_b1'], p['nc_w2'], p['nc_b2'],
      w1a, w1b, p['gcs_w'])


def _conv_pool_body(neigh_ref, hs_ref, gcr_ref, gcrb_ref,
                    conv_ref, psum_ref, pmax_ref):
    i = pl.program_id(0)
    neigh = jnp.concatenate([neigh_ref[0], neigh_ref[1]], axis=-1)
    conv = (jnp.dot(neigh, gcr_ref[...], preferred_element_type=jnp.float32)
            + gcrb_ref[...] + hs_ref[...])
    conv_ref[...] = conv
    bsum = jnp.sum(conv, axis=0, keepdims=True)
    bmax = jnp.max(conv, axis=0, keepdims=True)

    @pl.when(i == 0)
    def _():
        psum_ref[...] = bsum
        pmax_ref[...] = bmax

    @pl.when(i > 0)
    def _():
        psum_ref[...] += bsum
        pmax_ref[...] = jnp.maximum(pmax_ref[...], bmax)


def _conv_pool(neigh, hs, p):
    return pl.pallas_call(
        _conv_pool_body,
        grid=(N // BN,),
        in_specs=[
            pl.BlockSpec((2, BN, 128), lambda i: (0, i, 0)),
            pl.BlockSpec((BN, H), lambda i: (i, 0)),
            pl.BlockSpec((H, H), lambda i: (0, 0)),
            pl.BlockSpec((H,), lambda i: (0,)),
        ],
        out_specs=[
            pl.BlockSpec((BN, H), lambda i: (i, 0)),
            pl.BlockSpec((1, H), lambda i: (0, 0)),
            pl.BlockSpec((1, H), lambda i: (0, 0)),
        ],
        out_shape=[
            jax.ShapeDtypeStruct((N, H), jnp.float32),
            jax.ShapeDtypeStruct((1, H), jnp.float32),
            jax.ShapeDtypeStruct((1, H), jnp.float32),
        ],
    )(neigh, hs, p['gcr_w'], p['gcr_b'])


def _graph_head_body(psum_ref, pmax_ref, opw_ref, opb_ref,
                     gw1_ref, gb1_ref, gw2_ref, gb2_ref, o_ref):
    add_p = psum_ref[...]
    mean_p = add_p / float(N)
    max_p = pmax_ref[...]
    ge = (jnp.dot(mean_p, opw_ref[0], preferred_element_type=jnp.float32)
          + jnp.dot(max_p, opw_ref[1], preferred_element_type=jnp.float32)
          + jnp.dot(add_p, opw_ref[2], preferred_element_type=jnp.float32)
          + opb_ref[...])
    t = jnp.maximum(jnp.dot(ge, gw1_ref[...], preferred_element_type=jnp.float32)
                    + gb1_ref[...], 0.0)
    o_ref[...] = (jnp.dot(t, gw2_ref[...], preferred_element_type=jnp.float32)
                  + gb2_ref[...])


def _graph_head(psum, pmax, p):
    opw = p['op_w'].reshape(3, H, H)
    return pl.pallas_call(
        _graph_head_body,
        out_shape=jax.ShapeDtypeStruct((1, 2), jnp.float32),
    )(psum, pmax, opw, p['op_b'], p['gc_w1'], p['gc_b1'], p['gc_w2'], p['gc_b2'])


def _edge_score_body(ab_ref, ea_ref, wc_ref, b1_ref, w2_ref, b2_ref, o_ref):
    u = (ab_ref[...]
         + jnp.dot(ea_ref[...], wc_ref[...], preferred_element_type=jnp.float32)
         + b1_ref[...])
    u = jnp.maximum(u, 0.0)
    s = jnp.dot(u, w2_ref[...], preferred_element_type=jnp.float32) + b2_ref[...]
    o_ref[...] = jax.nn.sigmoid(s)


def _edge_scores(ab, edge_attr, p):
    wc = p['ea_w1'][2 * H:]
    return pl.pallas_call(
        _edge_score_body,
        grid=(E // BE,),
        in_specs=[
            pl.BlockSpec((BE, H), lambda i: (i, 0)),
            pl.BlockSpec((BE, EDGE_F), lambda i: (i, 0)),
            pl.BlockSpec((EDGE_F, H), lambda i: (0, 0)),
            pl.BlockSpec((H,), lambda i: (0,)),
            pl.BlockSpec((H, 1), lambda i: (0, 0)),
            pl.BlockSpec((1,), lambda i: (0,)),
        ],
        out_specs=pl.BlockSpec((BE, 1), lambda i: (i, 0)),
        out_shape=jax.ShapeDtypeStruct((E, 1), jnp.float32),
    )(ab, edge_attr, wc, p['ea_b1'], p['ea_w2'], p['ea_b2'])


_NC = 2
_NS = 16
_NW = _NC * _NS


def _wid():
    return lax.axis_index("s") * _NC + lax.axis_index("c")


_AB_CH = 80
_AB_PER_W = E // _NW


@functools.cache
def _get_sc_ab():
    mesh = plsc.VectorSubcoreMesh(core_axis_name="c", subcore_axis_name="s")

    @functools.partial(
        pl.kernel, mesh=mesh,
        out_type=jax.ShapeDtypeStruct((E, H), jnp.float32),
        scratch_types=[
            pltpu.VMEM((_AB_CH,), jnp.int32),
            pltpu.VMEM((_AB_CH,), jnp.int32),
            pltpu.VMEM((_AB_CH, H), jnp.float32),
            pltpu.VMEM((_AB_CH, H), jnp.float32),
            pltpu.SemaphoreType.DMA,
            pltpu.SemaphoreType.DMA,
        ],
    )
    def _sc_ab(a_hbm, b_hbm, src_hbm, dst_hbm, out_hbm, sidx, didx, abuf,
               bbuf, sem1, sem2):
        w = _wid()

        def chunk(g, _):
            base = w * _AB_PER_W + g * _AB_CH
            pltpu.sync_copy(src_hbm.at[pl.ds(base, _AB_CH)], sidx)
            pltpu.sync_copy(dst_hbm.at[pl.ds(base, _AB_CH)], didx)
            ca = pltpu.async_copy(a_hbm.at[sidx], abuf, sem1)
            cb = pltpu.async_copy(b_hbm.at[didx], bbuf, sem2)
            ca.wait()
            cb.wait()

            def row(j, _):
                for k in range(H // 16):
                    abuf[j, pl.ds(k * 16, 16)] = (
                        abuf[j, pl.ds(k * 16, 16)]
                        + bbuf[j, pl.ds(k * 16, 16)])
                return 0

            lax.fori_loop(0, _AB_CH, row, 0)
            pltpu.sync_copy(abuf, out_hbm.at[pl.ds(base, _AB_CH)])
            return 0

        lax.fori_loop(0, _AB_PER_W // _AB_CH, chunk, 0)

    return _sc_ab


_A_CH = 80
_A_PER_W = E // _NW
_DENP = 40960


@functools.cache
def _get_sc_edgew():
    mesh = plsc.VectorSubcoreMesh(core_axis_name="c", subcore_axis_name="s")

    @functools.partial(
        pl.kernel, mesh=mesh,
        out_type=[
            jax.ShapeDtypeStruct((HEADS * E,), jnp.float32),
            jax.ShapeDtypeStruct((_NC * _DENP,), jnp.float32),
        ],
        scratch_types=[
            pltpu.VMEM_SHARED((_DENP,), jnp.float32),
            pltpu.VMEM((_DENP // _NS,), jnp.float32),
            pltpu.VMEM((_A_CH,), jnp.int32),
            pltpu.VMEM((_A_CH,), jnp.int32),
            pltpu.VMEM((8, _A_CH), jnp.int32),
            pltpu.VMEM((8, _A_CH), jnp.float32),
            pltpu.VMEM((HEADS, _A_CH), jnp.float32),
            pltpu.VMEM((_A_CH,), jnp.int32),
            pltpu.SemaphoreType.DMA,
        ],
    )
    def _sc_edgew(sdt_hbm, src_hbm, dst_hbm, ee_hbm, den_hbm,
                  den_sp, zb, srcb, dstb, idxs, gsd, eebuf, idxd, sem):
        cidx = lax.axis_index("c")
        tid = lax.axis_index("s")
        w = tid * _NC + cidx

        per_tec = _DENP // _NS

        def zero(i, _):
            zb[pl.ds(i * 16, 16)] = jnp.zeros((16,), jnp.float32)
            return 0

        lax.fori_loop(0, per_tec // 16, zero, 0)
        pltpu.sync_copy(zb, den_sp.at[pl.ds(tid * per_tec, per_tec)])
        plsc.subcore_barrier()

        def chunk(g, _):
            base = w * _A_PER_W + g * _A_CH
            pltpu.sync_copy(src_hbm.at[pl.ds(base, _A_CH)], srcb)
            pltpu.sync_copy(dst_hbm.at[pl.ds(base, _A_CH)], dstb)

            def mkidx(q, _):
                sv = srcb[pl.ds(q * 16, 16)]
                dv = dstb[pl.ds(q * 16, 16)]
                for hd in range(HEADS):
                    idxs[hd, pl.ds(q * 16, 16)] = sv + (hd * N)
                    idxs[4 + hd, pl.ds(q * 16, 16)] = dv + ((4 + hd) * N)
                return 0

            lax.fori_loop(0, _A_CH // 16, mkidx, 0)
            copies = [pltpu.async_copy(sdt_hbm.at[idxs.at[r]], gsd.at[r], sem)
                      for r in range(8)]
            for cpy in copies:
                cpy.wait()

            def grp(q, _):
                for hd in range(HEADS):
                    e = (gsd[hd, pl.ds(q * 16, 16)]
                         + gsd[4 + hd, pl.ds(q * 16, 16)])
                    eebuf[hd, pl.ds(q * 16, 16)] = jnp.exp(
                        jnp.where(e > 0, e, 0.2 * e))
                return 0

            lax.fori_loop(0, _A_CH // 16, grp, 0)
            for hd in range(HEADS):
                pltpu.sync_copy(eebuf.at[hd],
                                ee_hbm.at[pl.ds(hd * E + base, _A_CH)])

            for hd in range(HEADS):
                def mkd(q, _, hd=hd):
                    idxd[pl.ds(q * 16, 16)] = (dstb[pl.ds(q * 16, 16)]
                                               + hd * N)
                    return 0

                lax.fori_loop(0, _A_CH // 16, mkd, 0)
                pltpu.sync_copy(eebuf.at[hd], den_sp.at[idxd], add=True)
            return 0

        lax.fori_loop(0, _A_PER_W // _A_CH, chunk, 0)
        plsc.subcore_barrier()
        pltpu.sync_copy(den_sp.at[pl.ds(tid * per_tec, per_tec)],
                        den_hbm.at[pl.ds(cidx * _DENP + tid * per_tec,
                                         per_tec)])

    return _sc_edgew


_B_CH = 80
_B_PER_S = E // _NS
_NPAD = 10240
_B_ZR = 128


@functools.cache
def _get_sc_agg():
    mesh = plsc.VectorSubcoreMesh(core_axis_name="c", subcore_axis_name="s")
    spc = 8 // _NC

    @functools.partial(
        pl.kernel, mesh=mesh,
        out_type=jax.ShapeDtypeStruct((8, _NPAD, 128), jnp.float32),
        scratch_types=[
            pltpu.VMEM_SHARED((_NPAD, 128), jnp.float32),
            pltpu.VMEM((_B_ZR, 128), jnp.float32),
            pltpu.VMEM((_B_CH, 128), jnp.float32),
            pltpu.VMEM((_B_CH,), jnp.int32),
            pltpu.VMEM((_B_CH,), jnp.int32),
            pltpu.VMEM((_B_CH,), jnp.int32),
            pltpu.VMEM((_B_CH,), jnp.float32),
            pltpu.SemaphoreType.DMA,
        ],
    )
    def _sc_agg(hw8_hbm, ee_hbm, src_hbm, dst_hbm, agg_hbm,
                acc, zbuf, ebuf, srcb, sidxb, dstb, eeb, sem):
        cidx = lax.axis_index("c")
        tid = lax.axis_index("s")

        def zrow(j, _):
            for k in range(8):
                zbuf[j, pl.ds(k * 16, 16)] = jnp.zeros((16,), jnp.float32)
            return 0

        lax.fori_loop(0, _B_ZR, zrow, 0)

        rows_per_tec = _NPAD // _NS
        row0 = tid * rows_per_tec

        for t in range(spc):
            sblk = cidx * spc + t
            hd = sblk // 2
            for q in range(rows_per_tec // _B_ZR):
                pltpu.sync_copy(zbuf, acc.at[pl.ds(row0 + q * _B_ZR, _B_ZR)])
            plsc.subcore_barrier()

            def chunk(g, _):
                base = tid * _B_PER_S + g * _B_CH
                pltpu.sync_copy(src_hbm.at[pl.ds(base, _B_CH)], srcb)
                pltpu.sync_copy(dst_hbm.at[pl.ds(base, _B_CH)], dstb)
                pltpu.sync_copy(ee_hbm.at[pl.ds(hd * E + base, _B_CH)], eeb)

                def mkidx(q2, _):
                    v = srcb[pl.ds(q2 * 16, 16)]
                    sidxb[pl.ds(q2 * 16, 16)] = v * 8 + sblk
                    return 0

                lax.fori_loop(0, _B_CH // 16, mkidx, 0)
                pltpu.async_copy(hw8_hbm.at[sidxb], ebuf, sem).wait()

                def grp(q2, _):
                    ee16 = eeb[pl.ds(q2 * 16, 16)]
                    for j in range(16):
                        wv = jnp.broadcast_to(lax.slice(ee16, (j,), (j + 1,)),
                                              (16,))
                        row = q2 * 16 + j
                        for r in range(8):
                            ebuf[row, pl.ds(r * 16, 16)] = (
                                ebuf[row, pl.ds(r * 16, 16)] * wv)
                    return 0

                lax.fori_loop(0, _B_CH // 16, grp, 0)
                pltpu.sync_copy(ebuf, acc.at[dstb], add=True)
                return 0

            lax.fori_loop(0, _B_PER_S // _B_CH, chunk, 0)
            plsc.subcore_barrier()
            for q in range(rows_per_tec // _B_ZR):
                r0 = row0 + q * _B_ZR
                pltpu.sync_copy(acc.at[pl.ds(r0, _B_ZR)],
                                agg_hbm.at[sblk, pl.ds(r0, _B_ZR)])
            plsc.subcore_barrier()

    return _sc_agg


@functools.cache
def _get_sc_neigh():
    mesh = plsc.VectorSubcoreMesh(core_axis_name="c", subcore_axis_name="s")

    @functools.partial(
        pl.kernel, mesh=mesh,
        out_type=jax.ShapeDtypeStruct((2, _NPAD, 128), jnp.float32),
        scratch_types=[
            pltpu.VMEM_SHARED((_NPAD, 128), jnp.float32),
            pltpu.VMEM((_B_ZR, 128), jnp.float32),
            pltpu.VMEM((_B_CH, 128), jnp.float32),
            pltpu.VMEM((_B_CH,), jnp.int32),
            pltpu.VMEM((_B_CH,), jnp.int32),
            pltpu.VMEM((_B_CH,), jnp.int32),
            pltpu.SemaphoreType.DMA,
        ],
    )
    def _sc_neigh(h2_hbm, src_hbm, dst_hbm, out_hbm,
                  acc, zbuf, ebuf, srcb, sidxb, dstb, sem):
        cidx = lax.axis_index("c")
        tid = lax.axis_index("s")

        def zrow(j, _):
            for k in range(8):
                zbuf[j, pl.ds(k * 16, 16)] = jnp.zeros((16,), jnp.float32)
            return 0

        lax.fori_loop(0, _B_ZR, zrow, 0)

        rows_per_tec = _NPAD // _NS
        row0 = tid * rows_per_tec
        sblk = cidx
        for q in range(rows_per_tec // _B_ZR):
            pltpu.sync_copy(zbuf, acc.at[pl.ds(row0 + q * _B_ZR, _B_ZR)])
        plsc.subcore_barrier()

        def chunk(g, _):
            base = tid * _B_PER_S + g * _B_CH
            pltpu.sync_copy(src_hbm.at[pl.ds(base, _B_CH)], srcb)
            pltpu.sync_copy(dst_hbm.at[pl.ds(base, _B_CH)], dstb)

            def mkidx(q2, _):
                v = srcb[pl.ds(q2 * 16, 16)]
                sidxb[pl.ds(q2 * 16, 16)] = v * 2 + sblk
                return 0

            lax.fori_loop(0, _B_CH // 16, mkidx, 0)
            pltpu.async_copy(h2_hbm.at[sidxb], ebuf, sem).wait()
            pltpu.sync_copy(ebuf, acc.at[dstb], add=True)
            return 0

        lax.fori_loop(0, _B_PER_S // _B_CH, chunk, 0)
        plsc.subcore_barrier()
        for q in range(rows_per_tec // _B_ZR):
            r0 = row0 + q * _B_ZR
            pltpu.sync_copy(acc.at[pl.ds(r0, _B_ZR)],
                            out_hbm.at[sblk, pl.ds(r0, _B_ZR)])
        plsc.subcore_barrier()

    return _sc_neigh


def kernel(x, edge_index, edge_attr, batch, params):
    p = params
    src = edge_index[0]
    dst = edge_index[1]

    h = _encoder(x, p)

    gat_concat = (True, True, False)
    for i in range(3):
        hw, sd = _gat_project(h, p['gat%d_w' % i], p['gat%d_as' % i],
                              p['gat%d_ad' % i])
        sdT = sd.T.reshape(-1)
        eeT, den_f = _get_sc_edgew()(sdT, src, dst)
        agg_b = _get_sc_agg()(hw.reshape(N * 8, 128), eeT, src, dst)
        den2 = (den_f.reshape(_NC, _DENP)[:, :HEADS * N]
                .reshape(_NC, HEADS, N).transpose(0, 2, 1))
        h = _gat_post(agg_b, den2, hw, sd,
                      p['gat%d_b' % i], p['nrm%d_g' % i],
                      p['nrm%d_b' % i], gat_concat[i])

    cp, node_logits, A, B, hs = _final_dense(h, p)

    neigh_b = _get_sc_neigh()(h.reshape(N * 2, 128), src, dst)
    conv, psum, pmax = _conv_pool(neigh_b, hs, p)
    graph_logits = _graph_head(psum, pmax, p)

    ab = _get_sc_ab()(A, B, src, dst)
    edge_scores = _edge_scores(ab, edge_attr, p)

    return (node_logits, graph_logits, edge_scores, cp)


# speedup vs baseline: 21.3089x; 1.9438x over previous
import functools

import jax
import jax.numpy as jnp
from jax import lax
from jax.experimental import pallas as pl
from jax.experimental.pallas import tpu as pltpu
from jax.experimental.pallas import tpu_sc as plsc

N = 10000
E = 320000
NODE_F = 128
EDGE_F = 16
H = 256
HEADS = 4
NUM_CLASSES = 10
NUM_CLUSTERS = 10

BN = 1000
BE = 2000


def _ln_rows(y, g, b):
    m = jnp.mean(y, axis=-1, keepdims=True)
    v = jnp.mean((y - m) ** 2, axis=-1, keepdims=True)
    return (y - m) * jax.lax.rsqrt(v + 1e-5) * g + b


def _enc_body(x_ref, w1_ref, b1_ref, w2_ref, b2_ref, g_ref, be_ref, o_ref):
    h1 = jnp.maximum(
        jnp.dot(x_ref[...], w1_ref[...], preferred_element_type=jnp.float32)
        + b1_ref[...], 0.0)
    h2 = jnp.dot(h1, w2_ref[...], preferred_element_type=jnp.float32) + b2_ref[...]
    o_ref[...] = _ln_rows(h2, g_ref[...], be_ref[...])


def _encoder(x, p):
    return pl.pallas_call(
        _enc_body,
        grid=(N // BN,),
        in_specs=[
            pl.BlockSpec((BN, NODE_F), lambda i: (i, 0)),
            pl.BlockSpec((NODE_F, H), lambda i: (0, 0)),
            pl.BlockSpec((H,), lambda i: (0,)),
            pl.BlockSpec((H, H), lambda i: (0, 0)),
            pl.BlockSpec((H,), lambda i: (0,)),
            pl.BlockSpec((H,), lambda i: (0,)),
            pl.BlockSpec((H,), lambda i: (0,)),
        ],
        out_specs=pl.BlockSpec((BN, H), lambda i: (i, 0)),
        out_shape=jax.ShapeDtypeStruct((N, H), jnp.float32),
    )(x, p['ne_w1'], p['ne_b1'], p['ne_w2'], p['ne_b2'], p['ne_g'], p['ne_be'])


def _gat_proj_body(h_ref, w_ref, as_ref, ad_ref, hw_ref, sd_ref):
    hw = jnp.dot(h_ref[...], w_ref[...], preferred_element_type=jnp.float32)
    hw_ref[...] = hw
    cols = []
    for hd in range(HEADS):
        blk = hw[:, hd * H:(hd + 1) * H]
        cols.append(jnp.sum(blk * as_ref[hd, :], axis=-1, keepdims=True))
    for hd in range(HEADS):
        blk = hw[:, hd * H:(hd + 1) * H]
        cols.append(jnp.sum(blk * ad_ref[hd, :], axis=-1, keepdims=True))
    sd_ref[...] = jnp.concatenate(cols, axis=-1)


def _gat_project(h, w, a_s, a_d):
    k = h.shape[1]
    return pl.pallas_call(
        _gat_proj_body,
        grid=(N // BN,),
        in_specs=[
            pl.BlockSpec((BN, k), lambda i: (i, 0)),
            pl.BlockSpec((k, HEADS * H), lambda i: (0, 0)),
            pl.BlockSpec((HEADS, H), lambda i: (0, 0)),
            pl.BlockSpec((HEADS, H), lambda i: (0, 0)),
        ],
        out_specs=[
            pl.BlockSpec((BN, HEADS * H), lambda i: (i, 0)),
            pl.BlockSpec((BN, 2 * HEADS), lambda i: (i, 0)),
        ],
        out_shape=[
            jax.ShapeDtypeStruct((N, HEADS * H), jnp.float32),
            jax.ShapeDtypeStruct((N, 2 * HEADS), jnp.float32),
        ],
    )(h, w, a_s, a_d)


def _gat_post_body(agg_ref, den_ref, hw_ref, sd_ref, b_ref, g_ref, be_ref,
                   o_ref, *, concat):
    sd = sd_ref[...]
    e_self = sd[:, :HEADS] + sd[:, HEADS:]
    ee_self = jnp.exp(jnp.where(e_self > 0, e_self, 0.2 * e_self))
    hw = hw_ref[...]
    den = den_ref[0] + den_ref[1] + ee_self
    outs = []
    for hd in range(HEADS):
        agg_hd = jnp.concatenate([agg_ref[2 * hd], agg_ref[2 * hd + 1]],
                                 axis=-1)
        a = (agg_hd
             + ee_self[:, hd:hd + 1] * hw[:, hd * H:(hd + 1) * H])
        outs.append(a / (den[:, hd:hd + 1] + 1e-16))
    if concat:
        y = jnp.concatenate(outs, axis=-1) + b_ref[...]
    else:
        y = (outs[0] + outs[1] + outs[2] + outs[3]) * 0.25 + b_ref[...]
    y = _ln_rows(y, g_ref[...], be_ref[...])
    o_ref[...] = jnp.where(y > 0, y, jnp.exp(jnp.minimum(y, 0.0)) - 1.0)


def _gat_post(agg, den, hw, sd, b, g, be, concat):
    od = HEADS * H if concat else H
    return pl.pallas_call(
        functools.partial(_gat_post_body, concat=concat),
        grid=(N // BN,),
        in_specs=[
            pl.BlockSpec((8, BN, 128), lambda i: (0, i, 0)),
            pl.BlockSpec((_NC, BN, HEADS), lambda i: (0, i, 0)),
            pl.BlockSpec((BN, HEADS * H), lambda i: (i, 0)),
            pl.BlockSpec((BN, 2 * HEADS), lambda i: (i, 0)),
            pl.BlockSpec((od,), lambda i: (0,)),
            pl.BlockSpec((od,), lambda i: (0,)),
            pl.BlockSpec((od,), lambda i: (0,)),
        ],
        out_specs=pl.BlockSpec((BN, od), lambda i: (i, 0)),
        out_shape=jax.ShapeDtypeStruct((N, od), jnp.float32),
    )(agg, den, hw, sd, b, g, be)


def _final_dense_body(h_ref, caw1_ref, cab1_ref, caw2_ref, cab2_ref,
                      ncw1_ref, ncb1_ref, ncw2_ref, ncb2_ref,
                      w1a_ref, w1b_ref, gcs_ref,
                      cp_ref, nl_ref, a_ref, b_ref, hs_ref):
    h = h_ref[...]
    t = jnp.maximum(jnp.dot(h, caw1_ref[...], preferred_element_type=jnp.float32)
                    + cab1_ref[...], 0.0)
    logits = (jnp.dot(t, caw2_ref[...], preferred_element_type=jnp.float32)
              + cab2_ref[...])
    lm = jnp.max(logits, axis=-1, keepdims=True)
    el = jnp.exp(logits - lm)
    cp_ref[...] = el / jnp.sum(el, axis=-1, keepdims=True)
    t2 = jnp.maximum(jnp.dot(h, ncw1_ref[...], preferred_element_type=jnp.float32)
                     + ncb1_ref[...], 0.0)
    nl_ref[...] = (jnp.dot(t2, ncw2_ref[...], preferred_element_type=jnp.float32)
                   + ncb2_ref[...])
    a_ref[...] = jnp.dot(h, w1a_ref[...], preferred_element_type=jnp.float32)
    b_ref[...] = jnp.dot(h, w1b_ref[...], preferred_element_type=jnp.float32)
    hs_ref[...] = jnp.dot(h, gcs_ref[...], preferred_element_type=jnp.float32)


def _final_dense(h, p):
    w1a = p['ea_w1'][:H]
    w1b = p['ea_w1'][H:2 * H]
    return pl.pallas_call(
        _final_dense_body,
        grid=(N // BN,),
        in_specs=[
            pl.BlockSpec((BN, H), lambda i: (i, 0)),
            pl.BlockSpec((H, H), lambda i: (0, 0)),
            pl.BlockSpec((H,), lambda i: (0,)),
            pl.BlockSpec((H, NUM_CLUSTERS), lambda i: (0, 0)),
            pl.BlockSpec((NUM_CLUSTERS,), lambda i: (0,)),
            pl.BlockSpec((H, H // 2), lambda i: (0, 0)),
            pl.BlockSpec((H // 2,), lambda i: (0,)),
            pl.BlockSpec((H // 2, NUM_CLASSES), lambda i: (0, 0)),
            pl.BlockSpec((NUM_CLASSES,), lambda i: (0,)),
            pl.BlockSpec((H, H), lambda i: (0, 0)),
            pl.BlockSpec((H, H), lambda i: (0, 0)),
            pl.BlockSpec((H, H), lambda i: (0, 0)),
        ],
        out_specs=[
            pl.BlockSpec((BN, NUM_CLUSTERS), lambda i: (i, 0)),
            pl.BlockSpec((BN, NUM_CLASSES), lambda i: (i, 0)),
            pl.BlockSpec((BN, H), lambda i: (i, 0)),
            pl.BlockSpec((BN, H), lambda i: (i, 0)),
            pl.BlockSpec((BN, H), lambda i: (i, 0)),
        ],
        out_shape=[
            jax.ShapeDtypeStruct((N, NUM_CLUSTERS), jnp.float32),
            jax.ShapeDtypeStruct((N, NUM_CLASSES), jnp.float32),
            jax.ShapeDtypeStruct((N, H), jnp.float32),
            jax.ShapeDtypeStruct((N, H), jnp.float32),
            jax.ShapeDtypeStruct((N, H), jnp.float32),
        ],
    )(h, p['ca_w1'], p['ca_b1'], p['ca_w2'], p['ca_b2'],
      p['nc_w1'], p['nc_b1'], p['nc_w2'], p['nc_b2'],
      w1a, w1b, p['gcs_w'])


def _conv_pool_body(neigh_ref, hs_ref, gcr_ref, gcrb_ref,
                    conv_ref, psum_ref, pmax_ref):
    i = pl.program_id(0)
    neigh = jnp.concatenate([neigh_ref[0], neigh_ref[1]], axis=-1)
    conv = (jnp.dot(neigh, gcr_ref[...], preferred_element_type=jnp.float32)
            + gcrb_ref[...] + hs_ref[...])
    conv_ref[...] = conv
    bsum = jnp.sum(conv, axis=0, keepdims=True)
    bmax = jnp.max(conv, axis=0, keepdims=True)

    @pl.when(i == 0)
    def _():
        psum_ref[...] = bsum
        pmax_ref[...] = bmax

    @pl.when(i > 0)
    def _():
        psum_ref[...] += bsum
        pmax_ref[...] = jnp.maximum(pmax_ref[...], bmax)


def _conv_pool(neigh, hs, p):
    return pl.pallas_call(
        _conv_pool_body,
        grid=(N // BN,),
        in_specs=[
            pl.BlockSpec((2, BN, 128), lambda i: (0, i, 0)),
            pl.BlockSpec((BN, H), lambda i: (i, 0)),
            pl.BlockSpec((H, H), lambda i: (0, 0)),
            pl.BlockSpec((H,), lambda i: (0,)),
        ],
        out_specs=[
            pl.BlockSpec((BN, H), lambda i: (i, 0)),
            pl.BlockSpec((1, H), lambda i: (0, 0)),
            pl.BlockSpec((1, H), lambda i: (0, 0)),
        ],
        out_shape=[
            jax.ShapeDtypeStruct((N, H), jnp.float32),
            jax.ShapeDtypeStruct((1, H), jnp.float32),
            jax.ShapeDtypeStruct((1, H), jnp.float32),
        ],
    )(neigh, hs, p['gcr_w'], p['gcr_b'])


def _graph_head_body(psum_ref, pmax_ref, opw_ref, opb_ref,
                     gw1_ref, gb1_ref, gw2_ref, gb2_ref, o_ref):
    add_p = psum_ref[...]
    mean_p = add_p / float(N)
    max_p = pmax_ref[...]
    ge = (jnp.dot(mean_p, opw_ref[0], preferred_element_type=jnp.float32)
          + jnp.dot(max_p, opw_ref[1], preferred_element_type=jnp.float32)
          + jnp.dot(add_p, opw_ref[2], preferred_element_type=jnp.float32)
          + opb_ref[...])
    t = jnp.maximum(jnp.dot(ge, gw1_ref[...], preferred_element_type=jnp.float32)
                    + gb1_ref[...], 0.0)
    o_ref[...] = (jnp.dot(t, gw2_ref[...], preferred_element_type=jnp.float32)
                  + gb2_ref[...])


def _graph_head(psum, pmax, p):
    opw = p['op_w'].reshape(3, H, H)
    return pl.pallas_call(
        _graph_head_body,
        out_shape=jax.ShapeDtypeStruct((1, 2), jnp.float32),
    )(psum, pmax, opw, p['op_b'], p['gc_w1'], p['gc_b1'], p['gc_w2'], p['gc_b2'])


def _edge_score_body(ab_ref, ea_ref, wc_ref, b1_ref, w2_ref, b2_ref, o_ref):
    u = (ab_ref[...]
         + jnp.dot(ea_ref[...], wc_ref[...], preferred_element_type=jnp.float32)
         + b1_ref[...])
    u = jnp.maximum(u, 0.0)
    s = jnp.dot(u, w2_ref[...], preferred_element_type=jnp.float32) + b2_ref[...]
    o_ref[...] = jax.nn.sigmoid(s)


def _edge_scores(ab, edge_attr, p):
    wc = p['ea_w1'][2 * H:]
    return pl.pallas_call(
        _edge_score_body,
        grid=(E // BE,),
        in_specs=[
            pl.BlockSpec((BE, H), lambda i: (i, 0)),
            pl.BlockSpec((BE, EDGE_F), lambda i: (i, 0)),
            pl.BlockSpec((EDGE_F, H), lambda i: (0, 0)),
            pl.BlockSpec((H,), lambda i: (0,)),
            pl.BlockSpec((H, 1), lambda i: (0, 0)),
            pl.BlockSpec((1,), lambda i: (0,)),
        ],
        out_specs=pl.BlockSpec((BE, 1), lambda i: (i, 0)),
        out_shape=jax.ShapeDtypeStruct((E, 1), jnp.float32),
    )(ab, edge_attr, wc, p['ea_b1'], p['ea_w2'], p['ea_b2'])


_NC = 2
_NS = 16
_NW = _NC * _NS


def _wid():
    return lax.axis_index("s") * _NC + lax.axis_index("c")


_AB_CH = 80
_AB_PER_W = E // _NW


@functools.cache
def _get_sc_ab():
    mesh = plsc.VectorSubcoreMesh(core_axis_name="c", subcore_axis_name="s")

    @functools.partial(
        pl.kernel, mesh=mesh,
        out_type=jax.ShapeDtypeStruct((E, H), jnp.float32),
        scratch_types=[
            pltpu.VMEM((_AB_CH,), jnp.int32),
            pltpu.VMEM((_AB_CH,), jnp.int32),
            pltpu.VMEM((_AB_CH, H), jnp.float32),
            pltpu.VMEM((_AB_CH, H), jnp.float32),
            pltpu.SemaphoreType.DMA,
            pltpu.SemaphoreType.DMA,
        ],
    )
    def _sc_ab(a_hbm, b_hbm, src_hbm, dst_hbm, out_hbm, sidx, didx, abuf,
               bbuf, sem1, sem2):
        w = _wid()

        def chunk(g, _):
            base = w * _AB_PER_W + g * _AB_CH
            pltpu.sync_copy(src_hbm.at[pl.ds(base, _AB_CH)], sidx)
            pltpu.sync_copy(dst_hbm.at[pl.ds(base, _AB_CH)], didx)
            ca = pltpu.async_copy(a_hbm.at[sidx], abuf, sem1)
            cb = pltpu.async_copy(b_hbm.at[didx], bbuf, sem2)
            ca.wait()
            cb.wait()

            def row(j, _):
                for k in range(H // 16):
                    abuf[j, pl.ds(k * 16, 16)] = (
                        abuf[j, pl.ds(k * 16, 16)]
                        + bbuf[j, pl.ds(k * 16, 16)])
                return 0

            lax.fori_loop(0, _AB_CH, row, 0)
            pltpu.sync_copy(abuf, out_hbm.at[pl.ds(base, _AB_CH)])
            return 0

        lax.fori_loop(0, _AB_PER_W // _AB_CH, chunk, 0)

    return _sc_ab


_A_CH = 80
_A_PER_W = E // _NW
_DENP = 40960


@functools.cache
def _get_sc_edgew():
    mesh = plsc.VectorSubcoreMesh(core_axis_name="c", subcore_axis_name="s")

    @functools.partial(
        pl.kernel, mesh=mesh,
        out_type=[
            jax.ShapeDtypeStruct((HEADS * E,), jnp.float32),
            jax.ShapeDtypeStruct((_NC * _DENP,), jnp.float32),
        ],
        scratch_types=[
            pltpu.VMEM_SHARED((_DENP,), jnp.float32),
            pltpu.VMEM((_DENP // _NS,), jnp.float32),
            pltpu.VMEM((_A_CH,), jnp.int32),
            pltpu.VMEM((_A_CH,), jnp.int32),
            pltpu.VMEM((8, _A_CH), jnp.int32),
            pltpu.VMEM((8, _A_CH), jnp.float32),
            pltpu.VMEM((HEADS, _A_CH), jnp.float32),
            pltpu.VMEM((_A_CH,), jnp.int32),
            pltpu.SemaphoreType.DMA,
        ],
    )
    def _sc_edgew(sdt_hbm, src_hbm, dst_hbm, ee_hbm, den_hbm,
                  den_sp, zb, srcb, dstb, idxs, gsd, eebuf, idxd, sem):
        cidx = lax.axis_index("c")
        tid = lax.axis_index("s")
        w = tid * _NC + cidx

        per_tec = _DENP // _NS

        def zero(i, _):
            zb[pl.ds(i * 16, 16)] = jnp.zeros((16,), jnp.float32)
            return 0

        lax.fori_loop(0, per_tec // 16, zero, 0)
        pltpu.sync_copy(zb, den_sp.at[pl.ds(tid * per_tec, per_tec)])
        plsc.subcore_barrier()

        def chunk(g, _):
            base = w * _A_PER_W + g * _A_CH
            pltpu.sync_copy(src_hbm.at[pl.ds(base, _A_CH)], srcb)
            pltpu.sync_copy(dst_hbm.at[pl.ds(base, _A_CH)], dstb)

            def mkidx(q, _):
                sv = srcb[pl.ds(q * 16, 16)]
                dv = dstb[pl.ds(q * 16, 16)]
                for hd in range(HEADS):
                    idxs[hd, pl.ds(q * 16, 16)] = sv + (hd * N)
                    idxs[4 + hd, pl.ds(q * 16, 16)] = dv + ((4 + hd) * N)
                return 0

            lax.fori_loop(0, _A_CH // 16, mkidx, 0)
            copies = [pltpu.async_copy(sdt_hbm.at[idxs.at[r]], gsd.at[r], sem)
                      for r in range(8)]
            for cpy in copies:
                cpy.wait()

            def grp(q, _):
                for hd in range(HEADS):
                    e = (gsd[hd, pl.ds(q * 16, 16)]
                         + gsd[4 + hd, pl.ds(q * 16, 16)])
                    eebuf[hd, pl.ds(q * 16, 16)] = jnp.exp(
                        jnp.where(e > 0, e, 0.2 * e))
                return 0

            lax.fori_loop(0, _A_CH // 16, grp, 0)
            for hd in range(HEADS):
                pltpu.sync_copy(eebuf.at[hd],
                                ee_hbm.at[pl.ds(hd * E + base, _A_CH)])

            for hd in range(HEADS):
                def mkd(q, _, hd=hd):
                    idxd[pl.ds(q * 16, 16)] = (dstb[pl.ds(q * 16, 16)]
                                               + hd * N)
                    return 0

                lax.fori_loop(0, _A_CH // 16, mkd, 0)
                pltpu.sync_copy(eebuf.at[hd], den_sp.at[idxd], add=True)
            return 0

        lax.fori_loop(0, _A_PER_W // _A_CH, chunk, 0)
        plsc.subcore_barrier()
        pltpu.sync_copy(den_sp.at[pl.ds(tid * per_tec, per_tec)],
                        den_hbm.at[pl.ds(cidx * _DENP + tid * per_tec,
                                         per_tec)])

    return _sc_edgew


_B_CH = 80
_B_PER_S = E // _NS
_NPAD = 10240
_B_ZR = 128


@functools.cache
def _get_sc_agg():
    mesh = plsc.VectorSubcoreMesh(core_axis_name="c", subcore_axis_name="s")
    spc = 8 // _NC
    stg = 2000
    cpb = stg // _B_CH
    nblk = _B_PER_S // stg

    @functools.partial(
        pl.kernel, mesh=mesh,
        out_type=jax.ShapeDtypeStruct((8, _NPAD, 128), jnp.float32),
        scratch_types=[
            pltpu.VMEM_SHARED((_NPAD, 128), jnp.float32),
            pltpu.VMEM((_B_ZR, 128), jnp.float32),
            pltpu.VMEM((_B_CH, 128), jnp.float32),
            pltpu.VMEM((_B_CH, 128), jnp.float32),
            pltpu.VMEM((stg,), jnp.int32),
            pltpu.VMEM((stg,), jnp.float32),
            pltpu.VMEM((cpb, _B_CH), jnp.int32),
            pltpu.VMEM((cpb, _B_CH), jnp.int32),
            pltpu.SemaphoreType.DMA,
            pltpu.SemaphoreType.DMA,
        ],
    )
    def _sc_agg(hw8_hbm, ee_hbm, src_hbm, dst_hbm, agg_hbm,
                acc, zbuf, buf0, buf1, flatb, eeL, sidx, didx, sem0, sem1):
        cidx = lax.axis_index("c")
        tid = lax.axis_index("s")

        def zrow(j, _):
            for k in range(8):
                zbuf[j, pl.ds(k * 16, 16)] = jnp.zeros((16,), jnp.float32)
            return 0

        lax.fori_loop(0, _B_ZR, zrow, 0)

        rows_per_tec = _NPAD // _NS
        row0 = tid * rows_per_tec

        def scale(buf, r):
            def sgrp(q, _):
                ee16 = eeL[pl.ds(r * _B_CH + q * 16, 16)]
                for j in range(16):
                    wv = jnp.broadcast_to(
                        lax.slice(ee16, (j,), (j + 1,)), (16,))
                    row = q * 16 + j
                    for rr in range(8):
                        buf[row, pl.ds(rr * 16, 16)] = (
                            buf[row, pl.ds(rr * 16, 16)] * wv)
                return 0

            lax.fori_loop(0, _B_CH // 16, sgrp, 0)

        for t in range(spc):
            sblk = cidx * spc + t
            hd = sblk // 2
            for q in range(rows_per_tec // _B_ZR):
                pltpu.sync_copy(zbuf, acc.at[pl.ds(row0 + q * _B_ZR, _B_ZR)])
            plsc.subcore_barrier()

            def block(bk, _):
                bbase = tid * _B_PER_S + bk * stg
                pltpu.sync_copy(src_hbm.at[pl.ds(bbase, stg)], flatb)

                def mks(r, _):
                    for k in range(_B_CH // 16):
                        v = flatb[pl.ds(r * _B_CH + k * 16, 16)]
                        sidx[r, pl.ds(k * 16, 16)] = v * 8 + sblk
                    return 0

                lax.fori_loop(0, cpb, mks, 0)
                pltpu.sync_copy(dst_hbm.at[pl.ds(bbase, stg)], flatb)

                def mkd(r, _):
                    for k in range(_B_CH // 16):
                        didx[r, pl.ds(k * 16, 16)] = flatb[
                            pl.ds(r * _B_CH + k * 16, 16)]
                    return 0

                lax.fori_loop(0, cpb, mkd, 0)
                pltpu.sync_copy(ee_hbm.at[pl.ds(hd * E + bbase, stg)], eeL)

                pltpu.async_copy(hw8_hbm.at[sidx.at[0]], buf0, sem0)

                def pair(k, _):
                    r0 = 2 * k
                    d1 = pltpu.async_copy(hw8_hbm.at[sidx.at[r0 + 1]],
                                          buf1, sem1)
                    pltpu.make_async_copy(hw8_hbm.at[sidx.at[r0]],
                                          buf0, sem0).wait()
                    scale(buf0, r0)
                    pltpu.sync_copy(buf0, acc.at[didx.at[r0]], add=True)
                    pltpu.async_copy(hw8_hbm.at[sidx.at[r0 + 2]], buf0, sem0)
                    d1.wait()
                    scale(buf1, r0 + 1)
                    pltpu.sync_copy(buf1, acc.at[didx.at[r0 + 1]], add=True)
                    return 0

                lax.fori_loop(0, (cpb - 1) // 2, pair, 0)
                last = cpb - 1
                pltpu.make_async_copy(hw8_hbm.at[sidx.at[last]],
                                      buf0, sem0).wait()
                scale(buf0, last)
                pltpu.sync_copy(buf0, acc.at[didx.at[last]], add=True)
                return 0

            lax.fori_loop(0, nblk, block, 0)
            plsc.subcore_barrier()
            for q in range(rows_per_tec // _B_ZR):
                r0 = row0 + q * _B_ZR
                pltpu.sync_copy(acc.at[pl.ds(r0, _B_ZR)],
                                agg_hbm.at[sblk, pl.ds(r0, _B_ZR)])
            plsc.subcore_barrier()

    return _sc_agg


@functools.cache
def _get_sc_neigh():
    mesh = plsc.VectorSubcoreMesh(core_axis_name="c", subcore_axis_name="s")

    @functools.partial(
        pl.kernel, mesh=mesh,
        out_type=jax.ShapeDtypeStruct((2, _NPAD, 128), jnp.float32),
        scratch_types=[
            pltpu.VMEM_SHARED((_NPAD, 128), jnp.float32),
            pltpu.VMEM((_B_ZR, 128), jnp.float32),
            pltpu.VMEM((_B_CH, 128), jnp.float32),
            pltpu.VMEM((_B_CH,), jnp.int32),
            pltpu.VMEM((_B_CH,), jnp.int32),
            pltpu.VMEM((_B_CH,), jnp.int32),
            pltpu.SemaphoreType.DMA,
        ],
    )
    def _sc_neigh(h2_hbm, src_hbm, dst_hbm, out_hbm,
                  acc, zbuf, ebuf, srcb, sidxb, dstb, sem):
        cidx = lax.axis_index("c")
        tid = lax.axis_index("s")

        def zrow(j, _):
            for k in range(8):
                zbuf[j, pl.ds(k * 16, 16)] = jnp.zeros((16,), jnp.float32)
            return 0

        lax.fori_loop(0, _B_ZR, zrow, 0)

        rows_per_tec = _NPAD // _NS
        row0 = tid * rows_per_tec
        sblk = cidx
        for q in range(rows_per_tec // _B_ZR):
            pltpu.sync_copy(zbuf, acc.at[pl.ds(row0 + q * _B_ZR, _B_ZR)])
        plsc.subcore_barrier()

        def chunk(g, _):
            base = tid * _B_PER_S + g * _B_CH
            pltpu.sync_copy(src_hbm.at[pl.ds(base, _B_CH)], srcb)
            pltpu.sync_copy(dst_hbm.at[pl.ds(base, _B_CH)], dstb)

            def mkidx(q2, _):
                v = srcb[pl.ds(q2 * 16, 16)]
                sidxb[pl.ds(q2 * 16, 16)] = v * 2 + sblk
                return 0

            lax.fori_loop(0, _B_CH // 16, mkidx, 0)
            pltpu.async_copy(h2_hbm.at[sidxb], ebuf, sem).wait()
            pltpu.sync_copy(ebuf, acc.at[dstb], add=True)
            return 0

        lax.fori_loop(0, _B_PER_S // _B_CH, chunk, 0)
        plsc.subcore_barrier()
        for q in range(rows_per_tec // _B_ZR):
            r0 = row0 + q * _B_ZR
            pltpu.sync_copy(acc.at[pl.ds(r0, _B_ZR)],
                            out_hbm.at[sblk, pl.ds(r0, _B_ZR)])
        plsc.subcore_barrier()

    return _sc_neigh


def kernel(x, edge_index, edge_attr, batch, params):
    p = params
    src = edge_index[0]
    dst = edge_index[1]

    h = _encoder(x, p)

    gat_concat = (True, True, False)
    for i in range(3):
        hw, sd = _gat_project(h, p['gat%d_w' % i], p['gat%d_as' % i],
                              p['gat%d_ad' % i])
        sdT = sd.T.reshape(-1)
        eeT, den_f = _get_sc_edgew()(sdT, src, dst)
        agg_b = _get_sc_agg()(hw.reshape(N * 8, 128), eeT, src, dst)
        den2 = (den_f.reshape(_NC, _DENP)[:, :HEADS * N]
                .reshape(_NC, HEADS, N).transpose(0, 2, 1))
        h = _gat_post(agg_b, den2, hw, sd,
                      p['gat%d_b' % i], p['nrm%d_g' % i],
                      p['nrm%d_b' % i], gat_concat[i])

    cp, node_logits, A, B, hs = _final_dense(h, p)

    neigh_b = _get_sc_neigh()(h.reshape(N * 2, 128), src, dst)
    conv, psum, pmax = _conv_pool(neigh_b, hs, p)
    graph_logits = _graph_head(psum, pmax, p)

    ab = _get_sc_ab()(A, B, src, dst)
    edge_scores = _edge_scores(ab, edge_attr, p)

    return (node_logits, graph_logits, edge_scores, cp)
